# baseline (speedup 1.0000x reference)
import functools

import jax
import jax.numpy as jnp
from jax.experimental import pallas as pl
from jax.experimental.pallas import tpu as pltpu

_SIGMA = 1.0
_NCHUNK = 2


def _proxy_nca_body(xs_hbm, ys_ref, prox_ref, out_ref, buf0, buf1, sem0, sem1):
    B = xs_hbm.shape[0]
    C = prox_ref.shape[0]
    blk = B // _NCHUNK
    bufs, sems = (buf0, buf1), (sem0, sem1)

    copies = []
    for i in range(_NCHUNK):
        cp = pltpu.make_async_copy(
            xs_hbm.at[pl.ds(i * blk, blk), :], bufs[i], sems[i])
        cp.start()
        copies.append(cp)

    prox = prox_ref[:]
    n2 = jnp.sum(prox * prox, axis=1, keepdims=True)
    inv = 1.0 / jnp.maximum(jnp.sqrt(n2), 1e-12)
    pn2 = n2 * (inv * inv)
    two_inv = 2.0 * inv

    total = jnp.zeros((1, 1), jnp.float32)
    for i in range(_NCHUNK):
        copies[i].wait()
        raw = jax.lax.dot_general(
            prox, bufs[i][:], (((1,), (1,)), ((), ())),
            preferred_element_type=jnp.float32)
        s = (raw * two_inv - pn2) * (1.0 / _SIGMA)

        m = jnp.max(s, axis=0, keepdims=True)
        lse = m + jnp.log(jnp.sum(jnp.exp(s - m), axis=0, keepdims=True))

        row = jax.lax.broadcasted_iota(jnp.int32, (C, blk), 0)
        y_blk = ys_ref[:, i * blk:(i + 1) * blk]
        s_pos = jnp.sum(jnp.where(row == y_blk, s, 0.0),
                        axis=0, keepdims=True)
        total = total + jnp.sum(lse - s_pos, axis=(0, 1), keepdims=True)

    out_ref[:, :] = total * (1.0 / B)


@functools.partial(jax.jit, static_argnames=())
def kernel(xs, ys, proxies):
    B, D = xs.shape
    blk = B // _NCHUNK
    out = pl.pallas_call(
        _proxy_nca_body,
        in_specs=[
            pl.BlockSpec(memory_space=pl.ANY),
            pl.BlockSpec(memory_space=pltpu.VMEM),
            pl.BlockSpec(memory_space=pltpu.VMEM),
        ],
        out_specs=pl.BlockSpec(memory_space=pltpu.VMEM),
        out_shape=jax.ShapeDtypeStruct((1, 1), jnp.float32),
        scratch_shapes=[
            pltpu.VMEM((blk, D), jnp.float32),
            pltpu.VMEM((blk, D), jnp.float32),
            pltpu.SemaphoreType.DMA,
            pltpu.SemaphoreType.DMA,
        ],
    )(xs, ys.reshape(1, B), proxies)
    return out[0, 0]


# speedup vs baseline: 1.3796x; 1.3796x over previous
import functools

import jax
import jax.numpy as jnp
from jax.experimental import pallas as pl

_SIGMA = 1.0


def _proxy_nca_body(xs_ref, ys_ref, prox_ref, out_ref):
    i = pl.program_id(0)
    C = prox_ref.shape[0]
    blk = xs_ref.shape[0]
    nb = pl.num_programs(0)

    prox = prox_ref[:]
    n2 = jnp.sum(prox * prox, axis=1, keepdims=True)
    inv = 1.0 / jnp.maximum(jnp.sqrt(n2), 1e-12)
    pn2 = n2 * (inv * inv)

    raw = jax.lax.dot_general(
        prox, xs_ref[:], (((1,), (1,)), ((), ())),
        preferred_element_type=jnp.float32)
    s = (raw * (2.0 * inv) - pn2) * (1.0 / _SIGMA)

    m = jnp.max(s, axis=0, keepdims=True)
    lse = m + jnp.log(jnp.sum(jnp.exp(s - m), axis=0, keepdims=True))

    row = jax.lax.broadcasted_iota(jnp.int32, (C, blk), 0)
    s_pos = jnp.sum(jnp.where(row == ys_ref[:], s, 0.0),
                    axis=0, keepdims=True)
    part = jnp.sum(lse - s_pos, axis=(0, 1), keepdims=True) * (
        1.0 / (blk * nb))

    @pl.when(i == 0)
    def _init():
        out_ref[:, :] = jnp.zeros((1, 1), jnp.float32)

    out_ref[:, :] += part


@functools.partial(jax.jit, static_argnames=())
def kernel(xs, ys, proxies):
    B, D = xs.shape
    C = proxies.shape[0]
    nb = 2
    blk = B // nb
    out = pl.pallas_call(
        _proxy_nca_body,
        grid=(nb,),
        in_specs=[
            pl.BlockSpec((blk, D), lambda i: (i, 0)),
            pl.BlockSpec((1, blk), lambda i: (0, i)),
            pl.BlockSpec((C, D), lambda i: (0, 0)),
        ],
        out_specs=pl.BlockSpec((1, 1), lambda i: (0, 0)),
        out_shape=jax.ShapeDtypeStruct((1, 1), jnp.float32),
    )(xs, ys.reshape(1, B), proxies)
    return out[0, 0]


# speedup vs baseline: 1.4968x; 1.0849x over previous
import functools

import jax
import jax.numpy as jnp
from jax.experimental import pallas as pl

_SIGMA = 1.0


def _proxy_nca_body(xs_ref, ys_ref, prox_ref, out_ref):
    B = xs_ref.shape[0]
    C = prox_ref.shape[0]

    prox = prox_ref[:]
    raw = jax.lax.dot_general(
        prox, xs_ref[:], (((1,), (1,)), ((), ())),
        preferred_element_type=jnp.float32)
    n2 = jnp.sum(prox * prox, axis=1, keepdims=True)
    inv = 1.0 / jnp.maximum(jnp.sqrt(n2), 1e-12)
    pn2 = n2 * (inv * inv)
    s = (raw * (2.0 * inv) - pn2) * (1.0 / _SIGMA)

    m = jnp.max(s, axis=0, keepdims=True)
    lse = m + jnp.log(jnp.sum(jnp.exp(s - m), axis=0, keepdims=True))

    row = jax.lax.broadcasted_iota(jnp.int32, (C, B), 0)
    s_pos = jnp.sum(jnp.where(row == ys_ref[:], s, 0.0),
                    axis=0, keepdims=True)

    out_ref[:, :] = jnp.sum(lse - s_pos, axis=(0, 1), keepdims=True) * (1.0 / B)


@functools.partial(jax.jit, static_argnames=())
def kernel(xs, ys, proxies):
    out = pl.pallas_call(
        _proxy_nca_body,
        out_shape=jax.ShapeDtypeStruct((1, 1), jnp.float32),
    )(xs, ys.reshape(1, xs.shape[0]), proxies)
    return out[0, 0]
